# Initial kernel scaffold; baseline (speedup 1.0000x reference)
import jax
import jax.numpy as jnp
from jax.experimental import pallas as pl


def kernel(feat_stage3, feat_stage4, feat_stage5, intrinsics, extrinsics, params):
    raise NotImplementedError("write your pallas kernel here")


# speedup vs baseline: 1.3768x; 1.3768x over previous
import functools
import math

import jax
import jax.numpy as jnp
from jax import lax
from jax.experimental import pallas as pl
from jax.experimental.pallas import tpu as pltpu
from jax.experimental.pallas import tpu_sc as plsc

B = 6
BEV_H = 128
BEV_W = 128
NBINS = BEV_H * BEV_W
HALF = NBINS // 2
DEPTH_CH = 64
VOXEL = 0.8
_DEPTH_BINS = jnp.exp(jnp.linspace(0.0, math.log(35.0), DEPTH_CH)).astype(jnp.float32)

_S3 = (11264, 11264, 704, 64, 11)
_S4 = (2816, 3072, 192, 96, 2)
_S5 = (704, 768, 48, 48, 1)
_NTILES = 16
_ROWS_PER_TILE = HALF // _NTILES


def _sc_scatter_body(w3, i3, w4, i4, w5, i5, zeros,
                     out3, out4, out5,
                     acc, wb3, ib3, sx3, wb4, ib4, sx4, wb5, ib5, sx5):
    core = lax.axis_index("c")
    sid = lax.axis_index("s")
    base_bin = core * HALF
    row0 = sid * _ROWS_PER_TILE

    scales = (
        (w3, i3, out3, wb3, ib3, sx3, _S3),
        (w4, i4, out4, wb4, ib4, sx4, _S4),
        (w5, i5, out5, wb5, ib5, sx5, _S5),
    )

    def one_batch(b, carry):
        for (w, i, out, wb, ib, sx, (_, _, pt, ch, nch)) in scales:
            pltpu.sync_copy(zeros, acc.at[pl.ds(row0, _ROWS_PER_TILE)])
            plsc.subcore_barrier()

            def chunk_body(j, carry2, w=w, i=i, wb=wb, ib=ib, sx=sx,
                           pt=pt, ch=ch):
                off = sid * pt + j * ch
                pltpu.sync_copy(i.at[b, pl.ds(off, ch)], ib)
                pltpu.sync_copy(w.at[b, pl.ds(off, ch)], wb)
                for g in range(ch // 16):
                    v = ib[pl.ds(g * 16, 16)]
                    lo = v - base_bin
                    ok = (lo >= 0) & (lo < HALF)
                    sx[pl.ds(g * 16, 16)] = jnp.where(ok, lo, HALF)
                pltpu.sync_copy(wb, acc.at[sx], add=True)
                return carry2

            lax.fori_loop(0, nch, chunk_body, 0)
            plsc.subcore_barrier()
            pltpu.sync_copy(acc.at[pl.ds(row0, _ROWS_PER_TILE)],
                            out.at[b, pl.ds(base_bin + row0, _ROWS_PER_TILE)])
        return carry

    lax.fori_loop(0, B, one_batch, 0)


def _sc_scatter(w3, i3, w4, i4, w5, i5):
    zeros = jnp.zeros((_ROWS_PER_TILE, 128), jnp.float32)
    mesh = plsc.VectorSubcoreMesh(core_axis_name="c", subcore_axis_name="s")
    out = jax.ShapeDtypeStruct((B, NBINS, 128), jnp.float32)
    f = pl.kernel(
        _sc_scatter_body,
        out_type=(out, out, out),
        mesh=mesh,
        compiler_params=pltpu.CompilerParams(use_tc_tiling_on_sc=False),
        scratch_types=[
            pltpu.VMEM_SHARED((HALF + 8, 128), jnp.float32),
            pltpu.VMEM((_S3[3], 128), jnp.float32),
            pltpu.VMEM((_S3[3],), jnp.int32),
            pltpu.VMEM((_S3[3],), jnp.int32),
            pltpu.VMEM((_S4[3], 128), jnp.float32),
            pltpu.VMEM((_S4[3],), jnp.int32),
            pltpu.VMEM((_S4[3],), jnp.int32),
            pltpu.VMEM((_S5[3], 128), jnp.float32),
            pltpu.VMEM((_S5[3],), jnp.int32),
            pltpu.VMEM((_S5[3],), jnp.int32),
        ],
    )
    return f(w3, i3, w4, i4, w5, i5, zeros)


_BLK = 2048


def _fusion_body(x1, x2, x3, w1, w2, w3, b1, wo, bo, out):
    y = jnp.dot(x1[0], w1[...], preferred_element_type=jnp.float32)
    y += jnp.dot(x2[0], w2[...], preferred_element_type=jnp.float32)
    y += jnp.dot(x3[0], w3[...], preferred_element_type=jnp.float32)
    y = jnp.maximum(y + b1[...], 0.0)
    out[0] = jnp.dot(y, wo[...], preferred_element_type=jnp.float32) + bo[...]


def _fusion(x1, x2, x3, w1, w2, w3, b1, wo, bo):
    grid = (B, NBINS // _BLK)
    xspec = pl.BlockSpec((1, _BLK, 128), lambda b, i: (b, i, 0))
    wspec = pl.BlockSpec((128, 256), lambda b, i: (0, 0))
    return pl.pallas_call(
        _fusion_body,
        grid=grid,
        in_specs=[xspec, xspec, xspec,
                  wspec, wspec, wspec,
                  pl.BlockSpec((1, 256), lambda b, i: (0, 0)),
                  pl.BlockSpec((256, 128), lambda b, i: (0, 0)),
                  pl.BlockSpec((1, 128), lambda b, i: (0, 0))],
        out_specs=pl.BlockSpec((1, _BLK, 128), lambda b, i: (b, i, 0)),
        out_shape=jax.ShapeDtypeStruct((B, NBINS, 128), jnp.float32),
    )(x1, x2, x3, w1, w2, w3, b1, wo, bo)


def _conv(x, w, b=None, padding=0, groups=1):
    out = lax.conv_general_dilated(
        x, w, (1, 1), [(padding, padding), (padding, padding)],
        dimension_numbers=("NCHW", "OIHW", "NCHW"), feature_group_count=groups)
    if b is not None:
        out = out + b[None, :, None, None]
    return out


def _bn(x, g, b):
    s = g / jnp.float32(math.sqrt(1.0 + 1e-5))
    return x * s[None, :, None, None] + b[None, :, None, None]


def _head(f, p, K_inv, T, hw_pad):
    _, _, H, W = f.shape
    x = jnp.linspace(0.0, W - 1.0, W)
    y = jnp.linspace(0.0, H - 1.0, H)
    yy, xx = jnp.meshgrid(y, x, indexing="ij")
    grid = jnp.stack([xx, yy, jnp.ones_like(xx)], axis=-1).reshape(-1, 3).T

    h = jax.nn.relu(_bn(_conv(f, p["fr1_w"]), p["fr_bn1_g"], p["fr_bn1_b"]))
    reduced = jax.nn.relu(_bn(_conv(h, p["fr2_w"], padding=1, groups=8),
                              p["fr_bn2_g"], p["fr_bn2_b"]))
    d = jax.nn.relu(_bn(_conv(f, p["dn1_w"]), p["dn_bn_g"], p["dn_bn_b"]))
    depth_logits = _conv(d, p["dn2_w"], p["dn2_b"])
    depth_probs = jax.nn.softmax(depth_logits * 10.0, axis=1)
    depth_map = (depth_probs * _DEPTH_BINS[None, :, None, None]).sum(axis=1)
    c = jax.nn.relu(_bn(_conv(jnp.concatenate([depth_logits, reduced], axis=1),
                              p["cn1_w"], padding=1),
                        p["cn_bn_g"], p["cn_bn_b"]))
    confidence = jax.nn.sigmoid(_conv(c, p["cn2_w"], p["cn2_b"]))

    depth_flat = depth_map.reshape(B, 1, -1)
    cam_pts = depth_flat * jnp.matmul(K_inv, grid[None])
    cam_pts_h = jnp.concatenate([cam_pts, jnp.ones_like(cam_pts[:, :1])], axis=1)
    ego = jnp.matmul(T, cam_pts_h)[:, :3]
    bev_x = (ego[:, 0] / VOXEL + BEV_W // 2).astype(jnp.int32)
    bev_y = (ego[:, 1] / VOXEL + BEV_H // 2).astype(jnp.int32)
    valid = (bev_x >= 0) & (bev_x < BEV_W) & (bev_y >= 0) & (bev_y < BEV_H)
    weighted = reduced.reshape(B, 128, -1) * confidence.reshape(B, 1, -1)
    weighted = jnp.where(valid[:, None, :], weighted, 0.0)
    idx = jnp.where(valid, bev_y * BEV_W + bev_x, 0)

    wt = jnp.transpose(weighted, (0, 2, 1))
    hw = H * W
    if hw_pad > hw:
        wt = jnp.concatenate(
            [wt, jnp.zeros((B, hw_pad - hw, 128), wt.dtype)], axis=1)
        idx = jnp.concatenate(
            [idx, jnp.zeros((B, hw_pad - hw), jnp.int32)], axis=1)
    return wt, idx.astype(jnp.int32)


def kernel(feat_stage3, feat_stage4, feat_stage5, intrinsics, extrinsics, params):
    K_inv = jnp.linalg.inv(intrinsics)
    T = extrinsics.reshape(B, 4, 4)

    w3, i3 = _head(feat_stage3, params["stage3"], K_inv, T, _S3[1])
    w4, i4 = _head(feat_stage4, params["stage4"], K_inv, T, _S4[1])
    w5, i5 = _head(feat_stage5, params["stage5"], K_inv, T, _S5[1])

    bev3, bev4, bev5 = _sc_scatter(w3, i3, w4, i4, w5, i5)

    fp = params["fusion"]
    s = fp["fu_bn_g"] / jnp.float32(math.sqrt(1.0 + 1e-5))
    wcat = fp["fu1_w"][:, :, 0, 0].T * s[None, :]
    b1 = fp["fu_bn_b"].reshape(1, 256)
    wo = fp["fu2_w"][:, :, 0, 0].T
    bo = fp["fu2_b"].reshape(1, 128)

    out = _fusion(bev3, bev4, bev5,
                  wcat[0:128], wcat[128:256], wcat[256:384], b1, wo, bo)
    return jnp.transpose(out, (0, 2, 1)).reshape(B, 128, BEV_H, BEV_W)


# speedup vs baseline: 1.4440x; 1.0488x over previous
import functools
import math

import jax
import jax.numpy as jnp
from jax import lax
from jax.experimental import pallas as pl
from jax.experimental.pallas import tpu as pltpu
from jax.experimental.pallas import tpu_sc as plsc

B = 6
BEV_H = 128
BEV_W = 128
NBINS = BEV_H * BEV_W
HALF = NBINS // 2
DEPTH_CH = 64
VOXEL = 0.8
_DEPTH_BINS = jnp.exp(jnp.linspace(0.0, math.log(35.0), DEPTH_CH)).astype(jnp.float32)

_S3 = (11264, 11264, 704, 64, 11)
_S4 = (2816, 3072, 192, 96, 2)
_S5 = (704, 768, 48, 48, 1)
_NTILES = 16
_ROWS_PER_TILE = HALF // _NTILES


def _sc_scatter_body(w3, i3, w4, i4, w5, i5, zeros,
                     out3, out4, out5,
                     acc, wb3, ib3, sx3, wb4, ib4, sx4, wb5, ib5, sx5,
                     semz, sem0, sem1):
    core = lax.axis_index("c")
    sid = lax.axis_index("s")
    base_bin = core * HALF
    row0 = sid * _ROWS_PER_TILE

    scales = (
        (w3, i3, out3, wb3, ib3, sx3, _S3),
        (w4, i4, out4, wb4, ib4, sx4, _S4),
        (w5, i5, out5, wb5, ib5, sx5, _S5),
    )
    sems = (sem0, sem1)

    def one_batch(b, carry):
        for (w, i, out, wb, ib, sx, (_, _, pt, ch, nch)) in scales:
            az = pltpu.async_copy(zeros, acc.at[pl.ds(row0, _ROWS_PER_TILE)],
                                  semz)
            pltpu.sync_copy(i.at[b, pl.ds(sid * pt, pt)], ib)
            cps = [pltpu.async_copy(w.at[b, pl.ds(sid * pt, ch)], wb.at[0],
                                    sem0)]
            for g in range(pt // 16):
                v = ib[pl.ds(g * 16, 16)]
                lo = v - base_bin
                ok = (lo >= 0) & (lo < HALF)
                sx[(g * 16) // ch, pl.ds((g * 16) % ch, 16)] = (
                    jnp.where(ok, lo, HALF))
            az.wait()
            plsc.subcore_barrier()

            for j in range(nch):
                if j + 1 < nch:
                    cps.append(pltpu.async_copy(
                        w.at[b, pl.ds(sid * pt + (j + 1) * ch, ch)],
                        wb.at[(j + 1) % 2], sems[(j + 1) % 2]))
                cps[j].wait()
                pltpu.sync_copy(wb.at[j % 2], acc.at[sx.at[j]], add=True)
            plsc.subcore_barrier()
            pltpu.sync_copy(acc.at[pl.ds(row0, _ROWS_PER_TILE)],
                            out.at[b, pl.ds(base_bin + row0, _ROWS_PER_TILE)])
        return carry

    lax.fori_loop(0, B, one_batch, 0)


def _sc_scatter(w3, i3, w4, i4, w5, i5):
    zeros = jnp.zeros((_ROWS_PER_TILE, 128), jnp.float32)
    mesh = plsc.VectorSubcoreMesh(core_axis_name="c", subcore_axis_name="s")
    out = jax.ShapeDtypeStruct((B, NBINS, 128), jnp.float32)
    f = pl.kernel(
        _sc_scatter_body,
        out_type=(out, out, out),
        mesh=mesh,
        compiler_params=pltpu.CompilerParams(use_tc_tiling_on_sc=False),
        scratch_types=[
            pltpu.VMEM_SHARED((HALF + 8, 128), jnp.float32),
            pltpu.VMEM((2, _S3[3], 128), jnp.float32),
            pltpu.VMEM((_S3[2],), jnp.int32),
            pltpu.VMEM((_S3[4], _S3[3]), jnp.int32),
            pltpu.VMEM((2, _S4[3], 128), jnp.float32),
            pltpu.VMEM((_S4[2],), jnp.int32),
            pltpu.VMEM((_S4[4], _S4[3]), jnp.int32),
            pltpu.VMEM((2, _S5[3], 128), jnp.float32),
            pltpu.VMEM((_S5[2],), jnp.int32),
            pltpu.VMEM((_S5[4], _S5[3]), jnp.int32),
            pltpu.SemaphoreType.DMA,
            pltpu.SemaphoreType.DMA,
            pltpu.SemaphoreType.DMA,
        ],
    )
    return f(w3, i3, w4, i4, w5, i5, zeros)


_BLK = 2048


def _fusion_body(x1, x2, x3, w1, w2, w3, b1, wo, bo, out):
    y = jnp.dot(x1[0], w1[...], preferred_element_type=jnp.float32)
    y += jnp.dot(x2[0], w2[...], preferred_element_type=jnp.float32)
    y += jnp.dot(x3[0], w3[...], preferred_element_type=jnp.float32)
    y = jnp.maximum(y + b1[...], 0.0)
    o = jnp.dot(y, wo[...], preferred_element_type=jnp.float32) + bo[...]
    out[0] = o.T


def _fusion(x1, x2, x3, w1, w2, w3, b1, wo, bo):
    grid = (B, NBINS // _BLK)
    xspec = pl.BlockSpec((1, _BLK, 128), lambda b, i: (b, i, 0))
    wspec = pl.BlockSpec((128, 256), lambda b, i: (0, 0))
    return pl.pallas_call(
        _fusion_body,
        grid=grid,
        in_specs=[xspec, xspec, xspec,
                  wspec, wspec, wspec,
                  pl.BlockSpec((1, 256), lambda b, i: (0, 0)),
                  pl.BlockSpec((256, 128), lambda b, i: (0, 0)),
                  pl.BlockSpec((1, 128), lambda b, i: (0, 0))],
        out_specs=pl.BlockSpec((1, 128, _BLK), lambda b, i: (b, 0, i)),
        out_shape=jax.ShapeDtypeStruct((B, 128, NBINS), jnp.float32),
    )(x1, x2, x3, w1, w2, w3, b1, wo, bo)


def _transpad(x, hw_pad, blk):
    bsz, _, hw = x.shape

    def body(xr, out):
        out[0, 0:hw] = xr[0].T
        if hw_pad > hw:
            out[0, hw:hw_pad] = jnp.zeros((hw_pad - hw, 128), jnp.float32)

    return pl.pallas_call(
        body,
        grid=(bsz,),
        in_specs=[pl.BlockSpec((1, 128, hw), lambda b: (b, 0, 0))],
        out_specs=pl.BlockSpec((1, hw_pad, 128), lambda b: (b, 0, 0)),
        out_shape=jax.ShapeDtypeStruct((bsz, hw_pad, 128), jnp.float32),
    )(x)


def _conv(x, w, b=None, padding=0, groups=1):
    out = lax.conv_general_dilated(
        x, w, (1, 1), [(padding, padding), (padding, padding)],
        dimension_numbers=("NCHW", "OIHW", "NCHW"), feature_group_count=groups)
    if b is not None:
        out = out + b[None, :, None, None]
    return out


def _bn(x, g, b):
    s = g / jnp.float32(math.sqrt(1.0 + 1e-5))
    return x * s[None, :, None, None] + b[None, :, None, None]


def _head(f, p, K_inv, T, hw_pad):
    _, _, H, W = f.shape
    x = jnp.linspace(0.0, W - 1.0, W)
    y = jnp.linspace(0.0, H - 1.0, H)
    yy, xx = jnp.meshgrid(y, x, indexing="ij")
    grid = jnp.stack([xx, yy, jnp.ones_like(xx)], axis=-1).reshape(-1, 3).T

    h = jax.nn.relu(_bn(_conv(f, p["fr1_w"]), p["fr_bn1_g"], p["fr_bn1_b"]))
    reduced = jax.nn.relu(_bn(_conv(h, p["fr2_w"], padding=1, groups=8),
                              p["fr_bn2_g"], p["fr_bn2_b"]))
    d = jax.nn.relu(_bn(_conv(f, p["dn1_w"]), p["dn_bn_g"], p["dn_bn_b"]))
    depth_logits = _conv(d, p["dn2_w"], p["dn2_b"])
    depth_probs = jax.nn.softmax(depth_logits * 10.0, axis=1)
    depth_map = (depth_probs * _DEPTH_BINS[None, :, None, None]).sum(axis=1)
    c = jax.nn.relu(_bn(_conv(jnp.concatenate([depth_logits, reduced], axis=1),
                              p["cn1_w"], padding=1),
                        p["cn_bn_g"], p["cn_bn_b"]))
    confidence = jax.nn.sigmoid(_conv(c, p["cn2_w"], p["cn2_b"]))

    depth_flat = depth_map.reshape(B, 1, -1)
    cam_pts = depth_flat * jnp.matmul(K_inv, grid[None])
    cam_pts_h = jnp.concatenate([cam_pts, jnp.ones_like(cam_pts[:, :1])], axis=1)
    ego = jnp.matmul(T, cam_pts_h)[:, :3]
    bev_x = (ego[:, 0] / VOXEL + BEV_W // 2).astype(jnp.int32)
    bev_y = (ego[:, 1] / VOXEL + BEV_H // 2).astype(jnp.int32)
    valid = (bev_x >= 0) & (bev_x < BEV_W) & (bev_y >= 0) & (bev_y < BEV_H)
    weighted = reduced.reshape(B, 128, -1) * confidence.reshape(B, 1, -1)
    weighted = jnp.where(valid[:, None, :], weighted, 0.0)
    idx = jnp.where(valid, bev_y * BEV_W + bev_x, 0)

    blk = 64 if H * W % 256 else 256
    wt = _transpad(weighted, hw_pad, blk)
    hw = H * W
    if hw_pad > hw:
        idx = jnp.concatenate(
            [idx, jnp.zeros((B, hw_pad - hw), jnp.int32)], axis=1)
    return wt, idx.astype(jnp.int32)


def kernel(feat_stage3, feat_stage4, feat_stage5, intrinsics, extrinsics, params):
    K_inv = jnp.linalg.inv(intrinsics)
    T = extrinsics.reshape(B, 4, 4)

    w3, i3 = _head(feat_stage3, params["stage3"], K_inv, T, _S3[1])
    w4, i4 = _head(feat_stage4, params["stage4"], K_inv, T, _S4[1])
    w5, i5 = _head(feat_stage5, params["stage5"], K_inv, T, _S5[1])

    bev3, bev4, bev5 = _sc_scatter(w3, i3, w4, i4, w5, i5)

    fp = params["fusion"]
    s = fp["fu_bn_g"] / jnp.float32(math.sqrt(1.0 + 1e-5))
    wcat = fp["fu1_w"][:, :, 0, 0].T * s[None, :]
    b1 = fp["fu_bn_b"].reshape(1, 256)
    wo = fp["fu2_w"][:, :, 0, 0].T
    bo = fp["fu2_b"].reshape(1, 128)

    out = _fusion(bev3, bev4, bev5,
                  wcat[0:128], wcat[128:256], wcat[256:384], b1, wo, bo)
    return out.reshape(B, 128, BEV_H, BEV_W)


# speedup vs baseline: 1.4717x; 1.0192x over previous
import functools
import math

import jax
import jax.numpy as jnp
from jax import lax
from jax.experimental import pallas as pl
from jax.experimental.pallas import tpu as pltpu
from jax.experimental.pallas import tpu_sc as plsc

B = 6
BEV_H = 128
BEV_W = 128
NBINS = BEV_H * BEV_W
HALF = NBINS // 2
DEPTH_CH = 64
VOXEL = 0.8
_DEPTH_BINS = jnp.exp(jnp.linspace(0.0, math.log(35.0), DEPTH_CH)).astype(jnp.float32)

_S3 = (11264, 11264, 704, 64, 11)
_S4 = (2816, 3072, 192, 96, 2)
_S5 = (704, 768, 48, 48, 1)


def _ptp(pt):
    return (pt + 127) // 128
_NTILES = 16
_ROWS_PER_TILE = HALF // _NTILES


def _sc_scatter_body(w3, i3, w4, i4, w5, i5, zeros,
                     out3, out4, out5,
                     acc, wb3, ib3, sx3, wb4, ib4, sx4, wb5, ib5, sx5,
                     semz, sem0, sem1):
    core = lax.axis_index("c")
    sid = lax.axis_index("s")
    base_bin = core * HALF
    row0 = sid * _ROWS_PER_TILE

    scales = (
        (w3, i3, out3, wb3, ib3, sx3, _S3),
        (w4, i4, out4, wb4, ib4, sx4, _S4),
        (w5, i5, out5, wb5, ib5, sx5, _S5),
    )
    sems = (sem0, sem1)

    def one_batch(b, carry):
        for (w, i, out, wb, ib, sx, (_, _, pt, ch, nch)) in scales:
            az = pltpu.async_copy(zeros, acc.at[pl.ds(row0, _ROWS_PER_TILE)],
                                  semz)
            pltpu.sync_copy(i.at[b, sid], ib)
            cps = [pltpu.async_copy(w.at[b, pl.ds(sid * pt, ch)], wb.at[0],
                                    sem0)]
            for g in range(pt // 16):
                v = ib[(g * 16) // 128, pl.ds((g * 16) % 128, 16)]
                lo = v - base_bin
                ok = (lo >= 0) & (lo < HALF)
                sx[(g * 16) // ch, pl.ds((g * 16) % ch, 16)] = (
                    jnp.where(ok, lo, HALF))
            az.wait()
            plsc.subcore_barrier()

            for j in range(nch):
                if j + 1 < nch:
                    cps.append(pltpu.async_copy(
                        w.at[b, pl.ds(sid * pt + (j + 1) * ch, ch)],
                        wb.at[(j + 1) % 2], sems[(j + 1) % 2]))
                cps[j].wait()
                pltpu.sync_copy(wb.at[j % 2], acc.at[sx.at[j]], add=True)
            plsc.subcore_barrier()
            pltpu.sync_copy(acc.at[pl.ds(row0, _ROWS_PER_TILE)],
                            out.at[b, pl.ds(base_bin + row0, _ROWS_PER_TILE)])
        return carry

    lax.fori_loop(0, B, one_batch, 0)


def _sc_scatter(w3, i3, w4, i4, w5, i5):
    zeros = jnp.zeros((_ROWS_PER_TILE, 128), jnp.float32)

    def pack_idx(i, pt):
        rows = _ptp(pt)
        i = i.reshape(B, _NTILES, pt)
        i = jnp.pad(i, ((0, 0), (0, 0), (0, rows * 128 - pt)))
        return i.reshape(B, _NTILES, rows, 128)

    i3 = pack_idx(i3, _S3[2])
    i4 = pack_idx(i4, _S4[2])
    i5 = pack_idx(i5, _S5[2])
    mesh = plsc.VectorSubcoreMesh(core_axis_name="c", subcore_axis_name="s")
    out = jax.ShapeDtypeStruct((B, NBINS, 128), jnp.float32)
    f = pl.kernel(
        _sc_scatter_body,
        out_type=(out, out, out),
        mesh=mesh,
        compiler_params=pltpu.CompilerParams(use_tc_tiling_on_sc=True),
        scratch_types=[
            pltpu.VMEM_SHARED((HALF + 8, 128), jnp.float32),
            pltpu.VMEM((2, _S3[3], 128), jnp.float32),
            pltpu.VMEM((_ptp(_S3[2]), 128), jnp.int32),
            pltpu.VMEM((_S3[4], _S3[3]), jnp.int32),
            pltpu.VMEM((2, _S4[3], 128), jnp.float32),
            pltpu.VMEM((_ptp(_S4[2]), 128), jnp.int32),
            pltpu.VMEM((_S4[4], _S4[3]), jnp.int32),
            pltpu.VMEM((2, _S5[3], 128), jnp.float32),
            pltpu.VMEM((_ptp(_S5[2]), 128), jnp.int32),
            pltpu.VMEM((_S5[4], _S5[3]), jnp.int32),
            pltpu.SemaphoreType.DMA,
            pltpu.SemaphoreType.DMA,
            pltpu.SemaphoreType.DMA,
        ],
    )
    return f(w3, i3, w4, i4, w5, i5, zeros)


_BLK = 2048


def _fusion_body(x1, x2, x3, w1, w2, w3, b1, wo, bo, out):
    y = jnp.dot(x1[0], w1[...], preferred_element_type=jnp.float32)
    y += jnp.dot(x2[0], w2[...], preferred_element_type=jnp.float32)
    y += jnp.dot(x3[0], w3[...], preferred_element_type=jnp.float32)
    y = jnp.maximum(y + b1[...], 0.0)
    o = jnp.dot(y, wo[...], preferred_element_type=jnp.float32) + bo[...]
    out[0] = o.T.reshape(128, _BLK // BEV_W, BEV_W)


def _fusion(x1, x2, x3, w1, w2, w3, b1, wo, bo):
    grid = (B, NBINS // _BLK)
    xspec = pl.BlockSpec((1, _BLK, 128), lambda b, i: (b, i, 0))
    wspec = pl.BlockSpec((128, 256), lambda b, i: (0, 0))
    return pl.pallas_call(
        _fusion_body,
        grid=grid,
        in_specs=[xspec, xspec, xspec,
                  wspec, wspec, wspec,
                  pl.BlockSpec((1, 256), lambda b, i: (0, 0)),
                  pl.BlockSpec((256, 128), lambda b, i: (0, 0)),
                  pl.BlockSpec((1, 128), lambda b, i: (0, 0))],
        out_specs=pl.BlockSpec((1, 128, _BLK // BEV_W, BEV_W),
                               lambda b, i: (b, 0, i, 0)),
        out_shape=jax.ShapeDtypeStruct((B, 128, BEV_H, BEV_W), jnp.float32),
    )(x1, x2, x3, w1, w2, w3, b1, wo, bo)


def _transpad(x, hw_pad, blk):
    bsz, _, hw = x.shape

    def body(xr, out):
        out[0, 0:hw] = xr[0].T
        if hw_pad > hw:
            out[0, hw:hw_pad] = jnp.zeros((hw_pad - hw, 128), jnp.float32)

    return pl.pallas_call(
        body,
        grid=(bsz,),
        in_specs=[pl.BlockSpec((1, 128, hw), lambda b: (b, 0, 0))],
        out_specs=pl.BlockSpec((1, hw_pad, 128), lambda b: (b, 0, 0)),
        out_shape=jax.ShapeDtypeStruct((bsz, hw_pad, 128), jnp.float32),
    )(x)


def _conv(x, w, b=None, padding=0, groups=1):
    out = lax.conv_general_dilated(
        x, w, (1, 1), [(padding, padding), (padding, padding)],
        dimension_numbers=("NCHW", "OIHW", "NCHW"), feature_group_count=groups)
    if b is not None:
        out = out + b[None, :, None, None]
    return out


def _bn(x, g, b):
    s = g / jnp.float32(math.sqrt(1.0 + 1e-5))
    return x * s[None, :, None, None] + b[None, :, None, None]


def _head(f, p, K_inv, T, hw_pad):
    _, _, H, W = f.shape
    x = jnp.linspace(0.0, W - 1.0, W)
    y = jnp.linspace(0.0, H - 1.0, H)
    yy, xx = jnp.meshgrid(y, x, indexing="ij")
    grid = jnp.stack([xx, yy, jnp.ones_like(xx)], axis=-1).reshape(-1, 3).T

    h = jax.nn.relu(_bn(_conv(f, p["fr1_w"]), p["fr_bn1_g"], p["fr_bn1_b"]))
    reduced = jax.nn.relu(_bn(_conv(h, p["fr2_w"], padding=1, groups=8),
                              p["fr_bn2_g"], p["fr_bn2_b"]))
    d = jax.nn.relu(_bn(_conv(f, p["dn1_w"]), p["dn_bn_g"], p["dn_bn_b"]))
    depth_logits = _conv(d, p["dn2_w"], p["dn2_b"])
    depth_probs = jax.nn.softmax(depth_logits * 10.0, axis=1)
    depth_map = (depth_probs * _DEPTH_BINS[None, :, None, None]).sum(axis=1)
    c = jax.nn.relu(_bn(_conv(jnp.concatenate([depth_logits, reduced], axis=1),
                              p["cn1_w"], padding=1),
                        p["cn_bn_g"], p["cn_bn_b"]))
    confidence = jax.nn.sigmoid(_conv(c, p["cn2_w"], p["cn2_b"]))

    depth_flat = depth_map.reshape(B, 1, -1)
    cam_pts = depth_flat * jnp.matmul(K_inv, grid[None])
    cam_pts_h = jnp.concatenate([cam_pts, jnp.ones_like(cam_pts[:, :1])], axis=1)
    ego = jnp.matmul(T, cam_pts_h)[:, :3]
    bev_x = (ego[:, 0] / VOXEL + BEV_W // 2).astype(jnp.int32)
    bev_y = (ego[:, 1] / VOXEL + BEV_H // 2).astype(jnp.int32)
    valid = (bev_x >= 0) & (bev_x < BEV_W) & (bev_y >= 0) & (bev_y < BEV_H)
    weighted = reduced.reshape(B, 128, -1) * confidence.reshape(B, 1, -1)
    weighted = jnp.where(valid[:, None, :], weighted, 0.0)
    idx = jnp.where(valid, bev_y * BEV_W + bev_x, 0)

    blk = 64 if H * W % 256 else 256
    wt = _transpad(weighted, hw_pad, blk)
    hw = H * W
    if hw_pad > hw:
        idx = jnp.concatenate(
            [idx, jnp.zeros((B, hw_pad - hw), jnp.int32)], axis=1)
    return wt, idx.astype(jnp.int32)


def kernel(feat_stage3, feat_stage4, feat_stage5, intrinsics, extrinsics, params):
    K_inv = jnp.linalg.inv(intrinsics)
    T = extrinsics.reshape(B, 4, 4)

    w3, i3 = _head(feat_stage3, params["stage3"], K_inv, T, _S3[1])
    w4, i4 = _head(feat_stage4, params["stage4"], K_inv, T, _S4[1])
    w5, i5 = _head(feat_stage5, params["stage5"], K_inv, T, _S5[1])

    bev3, bev4, bev5 = _sc_scatter(w3, i3, w4, i4, w5, i5)

    fp = params["fusion"]
    s = fp["fu_bn_g"] / jnp.float32(math.sqrt(1.0 + 1e-5))
    wcat = fp["fu1_w"][:, :, 0, 0].T * s[None, :]
    b1 = fp["fu_bn_b"].reshape(1, 256)
    wo = fp["fu2_w"][:, :, 0, 0].T
    bo = fp["fu2_b"].reshape(1, 128)

    return _fusion(bev3, bev4, bev5,
                   wcat[0:128], wcat[128:256], wcat[256:384], b1, wo, bo)


# speedup vs baseline: 2.2328x; 1.5171x over previous
import functools
import math

import jax
import jax.numpy as jnp
from jax import lax
from jax.experimental import pallas as pl
from jax.experimental.pallas import tpu as pltpu
from jax.experimental.pallas import tpu_sc as plsc

B = 6
BEV_H = 128
BEV_W = 128
NBINS = BEV_H * BEV_W
HALF = NBINS // 2
DEPTH_CH = 64
VOXEL = 0.8
_DEPTH_BINS = jnp.exp(jnp.linspace(0.0, math.log(35.0), DEPTH_CH)).astype(jnp.float32)

_S3 = (11264, 11264, 704, 64, 11)
_S4 = (2816, 3072, 192, 96, 2)
_S5 = (704, 768, 48, 48, 1)


def _ptp(pt):
    return (pt + 127) // 128
_NTILES = 16
_ROWS_PER_TILE = HALF // _NTILES


def _sc_scatter_body(w3, i3, w4, i4, w5, i5, zeros,
                     out3, out4, out5,
                     acc, wb3, ib3, sx3, wb4, ib4, sx4, wb5, ib5, sx5,
                     semz, sem0, sem1):
    core = lax.axis_index("c")
    sid = lax.axis_index("s")
    base_bin = core * HALF
    row0 = sid * _ROWS_PER_TILE

    scales = (
        (w3, i3, out3, wb3, ib3, sx3, _S3),
        (w4, i4, out4, wb4, ib4, sx4, _S4),
        (w5, i5, out5, wb5, ib5, sx5, _S5),
    )
    sems = (sem0, sem1)

    def one_batch(b, carry):
        for (w, i, out, wb, ib, sx, (_, _, pt, ch, nch)) in scales:
            az = pltpu.async_copy(zeros, acc.at[pl.ds(row0, _ROWS_PER_TILE)],
                                  semz)
            pltpu.sync_copy(i.at[b, sid], ib)
            cps = [pltpu.async_copy(w.at[b, pl.ds(sid * pt, ch)], wb.at[0],
                                    sem0)]
            for g in range(pt // 16):
                v = ib[(g * 16) // 128, pl.ds((g * 16) % 128, 16)]
                lo = v - base_bin
                ok = (lo >= 0) & (lo < HALF)
                sx[(g * 16) // ch, pl.ds((g * 16) % ch, 16)] = (
                    jnp.where(ok, lo, HALF))
            az.wait()
            plsc.subcore_barrier()

            for j in range(nch):
                if j + 1 < nch:
                    cps.append(pltpu.async_copy(
                        w.at[b, pl.ds(sid * pt + (j + 1) * ch, ch)],
                        wb.at[(j + 1) % 2], sems[(j + 1) % 2]))
                cps[j].wait()
                pltpu.sync_copy(wb.at[j % 2], acc.at[sx.at[j]], add=True)
            plsc.subcore_barrier()
            pltpu.sync_copy(acc.at[pl.ds(row0, _ROWS_PER_TILE)],
                            out.at[b, pl.ds(base_bin + row0, _ROWS_PER_TILE)])
        return carry

    lax.fori_loop(0, B, one_batch, 0)


def _sc_scatter(w3, i3, w4, i4, w5, i5):
    zeros = jnp.zeros((_ROWS_PER_TILE, 128), jnp.float32)

    def pack_idx(i, pt):
        rows = _ptp(pt)
        i = i.reshape(B, _NTILES, pt)
        i = jnp.pad(i, ((0, 0), (0, 0), (0, rows * 128 - pt)))
        return i.reshape(B, _NTILES, rows, 128)

    i3 = pack_idx(i3, _S3[2])
    i4 = pack_idx(i4, _S4[2])
    i5 = pack_idx(i5, _S5[2])
    mesh = plsc.VectorSubcoreMesh(core_axis_name="c", subcore_axis_name="s")
    out = jax.ShapeDtypeStruct((B, NBINS, 128), jnp.float32)
    f = pl.kernel(
        _sc_scatter_body,
        out_type=(out, out, out),
        mesh=mesh,
        compiler_params=pltpu.CompilerParams(use_tc_tiling_on_sc=True),
        scratch_types=[
            pltpu.VMEM_SHARED((HALF + 8, 128), jnp.float32),
            pltpu.VMEM((2, _S3[3], 128), jnp.float32),
            pltpu.VMEM((_ptp(_S3[2]), 128), jnp.int32),
            pltpu.VMEM((_S3[4], _S3[3]), jnp.int32),
            pltpu.VMEM((2, _S4[3], 128), jnp.float32),
            pltpu.VMEM((_ptp(_S4[2]), 128), jnp.int32),
            pltpu.VMEM((_S4[4], _S4[3]), jnp.int32),
            pltpu.VMEM((2, _S5[3], 128), jnp.float32),
            pltpu.VMEM((_ptp(_S5[2]), 128), jnp.int32),
            pltpu.VMEM((_S5[4], _S5[3]), jnp.int32),
            pltpu.SemaphoreType.DMA,
            pltpu.SemaphoreType.DMA,
            pltpu.SemaphoreType.DMA,
        ],
    )
    return f(w3, i3, w4, i4, w5, i5, zeros)


_BLK = 2048


def _fusion_body(x1, x2, x3, w1, w2, w3, b1, wo, bo, out):
    y = jnp.dot(x1[0], w1[...], preferred_element_type=jnp.float32)
    y += jnp.dot(x2[0], w2[...], preferred_element_type=jnp.float32)
    y += jnp.dot(x3[0], w3[...], preferred_element_type=jnp.float32)
    y = jnp.maximum(y + b1[...], 0.0)
    o = jnp.dot(y, wo[...], preferred_element_type=jnp.float32) + bo[...]
    out[0] = o.T.reshape(128, _BLK // BEV_W, BEV_W)


def _fusion(x1, x2, x3, w1, w2, w3, b1, wo, bo):
    grid = (B, NBINS // _BLK)
    xspec = pl.BlockSpec((1, _BLK, 128), lambda b, i: (b, i, 0))
    wspec = pl.BlockSpec((128, 256), lambda b, i: (0, 0))
    return pl.pallas_call(
        _fusion_body,
        grid=grid,
        in_specs=[xspec, xspec, xspec,
                  wspec, wspec, wspec,
                  pl.BlockSpec((1, 256), lambda b, i: (0, 0)),
                  pl.BlockSpec((256, 128), lambda b, i: (0, 0)),
                  pl.BlockSpec((1, 128), lambda b, i: (0, 0))],
        out_specs=pl.BlockSpec((1, 128, _BLK // BEV_W, BEV_W),
                               lambda b, i: (b, 0, i, 0)),
        out_shape=jax.ShapeDtypeStruct((B, 128, BEV_H, BEV_W), jnp.float32),
    )(x1, x2, x3, w1, w2, w3, b1, wo, bo)


def _transpad(x, hw_pad, blk):
    bsz, _, hw = x.shape

    def body(xr, out):
        out[0, 0:hw] = xr[0].T
        if hw_pad > hw:
            out[0, hw:hw_pad] = jnp.zeros((hw_pad - hw, 128), jnp.float32)

    return pl.pallas_call(
        body,
        grid=(bsz,),
        in_specs=[pl.BlockSpec((1, 128, hw), lambda b: (b, 0, 0))],
        out_specs=pl.BlockSpec((1, hw_pad, 128), lambda b: (b, 0, 0)),
        out_shape=jax.ShapeDtypeStruct((bsz, hw_pad, 128), jnp.float32),
    )(x)


def _conv(x, w, b=None, padding=0, groups=1):
    out = lax.conv_general_dilated(
        x, w, (1, 1), [(padding, padding), (padding, padding)],
        dimension_numbers=("NCHW", "OIHW", "NCHW"), feature_group_count=groups)
    if b is not None:
        out = out + b[None, :, None, None]
    return out


def _bn(x, g, b):
    s = g / jnp.float32(math.sqrt(1.0 + 1e-5))
    return x * s[None, :, None, None] + b[None, :, None, None]


def _head(f, p, K_inv, T, hw_pad):
    _, _, H, W = f.shape
    x = jnp.linspace(0.0, W - 1.0, W)
    y = jnp.linspace(0.0, H - 1.0, H)
    yy, xx = jnp.meshgrid(y, x, indexing="ij")
    grid = jnp.stack([xx, yy, jnp.ones_like(xx)], axis=-1).reshape(-1, 3).T

    h = jax.nn.relu(_bn(_conv(f, p["fr1_w"]), p["fr_bn1_g"], p["fr_bn1_b"]))
    gw = p["fr2_w"]
    o_group = jnp.arange(128) // 16
    i_group = jnp.arange(64) // 8
    mask = (o_group[:, None] == i_group[None, :]).astype(jnp.float32)
    gw_dense = (jnp.tile(gw, (1, 8, 1, 1))
                * mask[:, :, None, None])
    reduced = jax.nn.relu(_bn(_conv(h, gw_dense, padding=1),
                              p["fr_bn2_g"], p["fr_bn2_b"]))
    d = jax.nn.relu(_bn(_conv(f, p["dn1_w"]), p["dn_bn_g"], p["dn_bn_b"]))
    depth_logits = _conv(d, p["dn2_w"], p["dn2_b"])
    depth_probs = jax.nn.softmax(depth_logits * 10.0, axis=1)
    depth_map = (depth_probs * _DEPTH_BINS[None, :, None, None]).sum(axis=1)
    c = jax.nn.relu(_bn(_conv(jnp.concatenate([depth_logits, reduced], axis=1),
                              p["cn1_w"], padding=1),
                        p["cn_bn_g"], p["cn_bn_b"]))
    confidence = jax.nn.sigmoid(_conv(c, p["cn2_w"], p["cn2_b"]))

    depth_flat = depth_map.reshape(B, 1, -1)
    cam_pts = depth_flat * jnp.matmul(K_inv, grid[None])
    cam_pts_h = jnp.concatenate([cam_pts, jnp.ones_like(cam_pts[:, :1])], axis=1)
    ego = jnp.matmul(T, cam_pts_h)[:, :3]
    bev_x = (ego[:, 0] / VOXEL + BEV_W // 2).astype(jnp.int32)
    bev_y = (ego[:, 1] / VOXEL + BEV_H // 2).astype(jnp.int32)
    valid = (bev_x >= 0) & (bev_x < BEV_W) & (bev_y >= 0) & (bev_y < BEV_H)
    weighted = reduced.reshape(B, 128, -1) * confidence.reshape(B, 1, -1)
    weighted = jnp.where(valid[:, None, :], weighted, 0.0)
    idx = jnp.where(valid, bev_y * BEV_W + bev_x, 0)

    blk = 64 if H * W % 256 else 256
    wt = _transpad(weighted, hw_pad, blk)
    hw = H * W
    if hw_pad > hw:
        idx = jnp.concatenate(
            [idx, jnp.zeros((B, hw_pad - hw), jnp.int32)], axis=1)
    return wt, idx.astype(jnp.int32)


def kernel(feat_stage3, feat_stage4, feat_stage5, intrinsics, extrinsics, params):
    K_inv = jnp.linalg.inv(intrinsics)
    T = extrinsics.reshape(B, 4, 4)

    w3, i3 = _head(feat_stage3, params["stage3"], K_inv, T, _S3[1])
    w4, i4 = _head(feat_stage4, params["stage4"], K_inv, T, _S4[1])
    w5, i5 = _head(feat_stage5, params["stage5"], K_inv, T, _S5[1])

    bev3, bev4, bev5 = _sc_scatter(w3, i3, w4, i4, w5, i5)

    fp = params["fusion"]
    s = fp["fu_bn_g"] / jnp.float32(math.sqrt(1.0 + 1e-5))
    wcat = fp["fu1_w"][:, :, 0, 0].T * s[None, :]
    b1 = fp["fu_bn_b"].reshape(1, 256)
    wo = fp["fu2_w"][:, :, 0, 0].T
    bo = fp["fu2_b"].reshape(1, 128)

    return _fusion(bev3, bev4, bev5,
                   wcat[0:128], wcat[128:256], wcat[256:384], b1, wo, bo)


# speedup vs baseline: 2.4464x; 1.0957x over previous
import functools
import math

import jax
import jax.numpy as jnp
from jax import lax
from jax.experimental import pallas as pl
from jax.experimental.pallas import tpu as pltpu
from jax.experimental.pallas import tpu_sc as plsc

B = 6
BEV_H = 128
BEV_W = 128
NBINS = BEV_H * BEV_W
HALF = NBINS // 2
DEPTH_CH = 64
VOXEL = 0.8
_DEPTH_BINS = jnp.exp(jnp.linspace(0.0, math.log(35.0), DEPTH_CH)).astype(jnp.float32)

_S3 = (11264, 11264, 704, 64, 11)
_S4 = (2816, 3072, 192, 96, 2)
_S5 = (704, 768, 48, 48, 1)


def _ptp(pt):
    return (pt + 127) // 128
_NTILES = 16
_ROWS_PER_TILE = HALF // _NTILES


def _make_sc_body(pt, ch, nch):
    def body(w, i, zeros, out, acc, wb, ib, sx, semz, sem0, sem1):
        core = lax.axis_index("c")
        sid = lax.axis_index("s")
        base_bin = core * HALF
        row0 = sid * _ROWS_PER_TILE
        sems = (sem0, sem1)

        def one_batch(b, carry):
            az = pltpu.async_copy(zeros, acc.at[pl.ds(row0, _ROWS_PER_TILE)],
                                  semz)
            pltpu.sync_copy(i.at[b, sid], ib)
            cps = [pltpu.async_copy(w.at[b, pl.ds(sid * pt, ch)], wb.at[0],
                                    sem0)]
            for g in range(pt // 16):
                v = ib[(g * 16) // 128, pl.ds((g * 16) % 128, 16)]
                lo = v - base_bin
                ok = (lo >= 0) & (lo < HALF)
                sx[(g * 16) // ch, pl.ds((g * 16) % ch, 16)] = (
                    jnp.where(ok, lo, HALF))
            az.wait()
            plsc.subcore_barrier()

            for j in range(nch):
                if j + 1 < nch:
                    cps.append(pltpu.async_copy(
                        w.at[b, pl.ds(sid * pt + (j + 1) * ch, ch)],
                        wb.at[(j + 1) % 2], sems[(j + 1) % 2]))
                cps[j].wait()
                pltpu.sync_copy(wb.at[j % 2], acc.at[sx.at[j]], add=True)
            plsc.subcore_barrier()
            pltpu.sync_copy(acc.at[pl.ds(row0, _ROWS_PER_TILE)],
                            out.at[b, pl.ds(base_bin + row0, _ROWS_PER_TILE)])
            return carry

        lax.fori_loop(0, B, one_batch, 0)

    return body


def _sc_scatter_one(w, i, spec):
    _, _, pt, ch, nch = spec
    zeros = jnp.zeros((_ROWS_PER_TILE, 128), jnp.float32)
    rows = _ptp(pt)
    i = i.reshape(B, _NTILES, pt)
    i = jnp.pad(i, ((0, 0), (0, 0), (0, rows * 128 - pt)))
    i = i.reshape(B, _NTILES, rows, 128)
    mesh = plsc.VectorSubcoreMesh(core_axis_name="c", subcore_axis_name="s")
    f = pl.kernel(
        _make_sc_body(pt, ch, nch),
        out_type=jax.ShapeDtypeStruct((B, NBINS, 128), jnp.float32),
        mesh=mesh,
        compiler_params=pltpu.CompilerParams(use_tc_tiling_on_sc=True),
        scratch_types=[
            pltpu.VMEM_SHARED((HALF + 8, 128), jnp.float32),
            pltpu.VMEM((2, ch, 128), jnp.float32),
            pltpu.VMEM((rows, 128), jnp.int32),
            pltpu.VMEM((nch, ch), jnp.int32),
            pltpu.SemaphoreType.DMA,
            pltpu.SemaphoreType.DMA,
            pltpu.SemaphoreType.DMA,
        ],
    )
    return f(w, i, zeros)


_BLK = 2048


def _fusion_body(x1, x2, x3, w1, w2, w3, b1, wo, bo, out):
    y = jnp.dot(x1[0], w1[...], preferred_element_type=jnp.float32)
    y += jnp.dot(x2[0], w2[...], preferred_element_type=jnp.float32)
    y += jnp.dot(x3[0], w3[...], preferred_element_type=jnp.float32)
    y = jnp.maximum(y + b1[...], 0.0)
    o = jnp.dot(y, wo[...], preferred_element_type=jnp.float32) + bo[...]
    out[0] = o.T.reshape(128, _BLK // BEV_W, BEV_W)


def _fusion(x1, x2, x3, w1, w2, w3, b1, wo, bo):
    grid = (B, NBINS // _BLK)
    xspec = pl.BlockSpec((1, _BLK, 128), lambda b, i: (b, i, 0))
    wspec = pl.BlockSpec((128, 256), lambda b, i: (0, 0))
    return pl.pallas_call(
        _fusion_body,
        grid=grid,
        in_specs=[xspec, xspec, xspec,
                  wspec, wspec, wspec,
                  pl.BlockSpec((1, 256), lambda b, i: (0, 0)),
                  pl.BlockSpec((256, 128), lambda b, i: (0, 0)),
                  pl.BlockSpec((1, 128), lambda b, i: (0, 0))],
        out_specs=pl.BlockSpec((1, 128, _BLK // BEV_W, BEV_W),
                               lambda b, i: (b, 0, i, 0)),
        out_shape=jax.ShapeDtypeStruct((B, 128, BEV_H, BEV_W), jnp.float32),
    )(x1, x2, x3, w1, w2, w3, b1, wo, bo)


def _transpad(x, hw_pad, blk):
    bsz, _, hw = x.shape

    def body(xr, out):
        out[0, 0:hw] = xr[0].T
        if hw_pad > hw:
            out[0, hw:hw_pad] = jnp.zeros((hw_pad - hw, 128), jnp.float32)

    return pl.pallas_call(
        body,
        grid=(bsz,),
        in_specs=[pl.BlockSpec((1, 128, hw), lambda b: (b, 0, 0))],
        out_specs=pl.BlockSpec((1, hw_pad, 128), lambda b: (b, 0, 0)),
        out_shape=jax.ShapeDtypeStruct((bsz, hw_pad, 128), jnp.float32),
    )(x)


def _conv(x, w, b=None, padding=0, groups=1):
    out = lax.conv_general_dilated(
        x, w, (1, 1), [(padding, padding), (padding, padding)],
        dimension_numbers=("NCHW", "OIHW", "NCHW"), feature_group_count=groups)
    if b is not None:
        out = out + b[None, :, None, None]
    return out


def _bn(x, g, b):
    s = g / jnp.float32(math.sqrt(1.0 + 1e-5))
    return x * s[None, :, None, None] + b[None, :, None, None]


def _head(f, p, K_inv, T, hw_pad):
    _, _, H, W = f.shape
    x = jnp.linspace(0.0, W - 1.0, W)
    y = jnp.linspace(0.0, H - 1.0, H)
    yy, xx = jnp.meshgrid(y, x, indexing="ij")
    grid = jnp.stack([xx, yy, jnp.ones_like(xx)], axis=-1).reshape(-1, 3).T

    h = jax.nn.relu(_bn(_conv(f, p["fr1_w"]), p["fr_bn1_g"], p["fr_bn1_b"]))
    gw = p["fr2_w"]
    o_group = jnp.arange(128) // 16
    i_group = jnp.arange(64) // 8
    mask = (o_group[:, None] == i_group[None, :]).astype(jnp.float32)
    gw_dense = (jnp.tile(gw, (1, 8, 1, 1))
                * mask[:, :, None, None])
    reduced = jax.nn.relu(_bn(_conv(h, gw_dense, padding=1),
                              p["fr_bn2_g"], p["fr_bn2_b"]))
    d = jax.nn.relu(_bn(_conv(f, p["dn1_w"]), p["dn_bn_g"], p["dn_bn_b"]))
    depth_logits = _conv(d, p["dn2_w"], p["dn2_b"])
    depth_probs = jax.nn.softmax(depth_logits * 10.0, axis=1)
    depth_map = (depth_probs * _DEPTH_BINS[None, :, None, None]).sum(axis=1)
    c = jax.nn.relu(_bn(_conv(jnp.concatenate([depth_logits, reduced], axis=1),
                              p["cn1_w"], padding=1),
                        p["cn_bn_g"], p["cn_bn_b"]))
    confidence = jax.nn.sigmoid(_conv(c, p["cn2_w"], p["cn2_b"]))

    depth_flat = depth_map.reshape(B, 1, -1)
    cam_pts = depth_flat * jnp.matmul(K_inv, grid[None])
    cam_pts_h = jnp.concatenate([cam_pts, jnp.ones_like(cam_pts[:, :1])], axis=1)
    ego = jnp.matmul(T, cam_pts_h)[:, :3]
    bev_x = (ego[:, 0] / VOXEL + BEV_W // 2).astype(jnp.int32)
    bev_y = (ego[:, 1] / VOXEL + BEV_H // 2).astype(jnp.int32)
    valid = (bev_x >= 0) & (bev_x < BEV_W) & (bev_y >= 0) & (bev_y < BEV_H)
    weighted = reduced.reshape(B, 128, -1) * confidence.reshape(B, 1, -1)
    weighted = jnp.where(valid[:, None, :], weighted, 0.0)
    idx = jnp.where(valid, bev_y * BEV_W + bev_x, 0)

    blk = 64 if H * W % 256 else 256
    wt = _transpad(weighted, hw_pad, blk)
    hw = H * W
    if hw_pad > hw:
        idx = jnp.concatenate(
            [idx, jnp.zeros((B, hw_pad - hw), jnp.int32)], axis=1)
    return wt, idx.astype(jnp.int32)


def kernel(feat_stage3, feat_stage4, feat_stage5, intrinsics, extrinsics, params):
    K_inv = jnp.linalg.inv(intrinsics)
    T = extrinsics.reshape(B, 4, 4)

    w5, i5 = _head(feat_stage5, params["stage5"], K_inv, T, _S5[1])
    bev5 = _sc_scatter_one(w5, i5, _S5)
    w4, i4 = _head(feat_stage4, params["stage4"], K_inv, T, _S4[1])
    i4 = i4 + bev5[0, 0, 0].astype(jnp.int32) * 0
    bev4 = _sc_scatter_one(w4, i4, _S4)
    w3, i3 = _head(feat_stage3, params["stage3"], K_inv, T, _S3[1])
    i3 = i3 + bev4[0, 0, 0].astype(jnp.int32) * 0
    bev3 = _sc_scatter_one(w3, i3, _S3)

    fp = params["fusion"]
    s = fp["fu_bn_g"] / jnp.float32(math.sqrt(1.0 + 1e-5))
    wcat = fp["fu1_w"][:, :, 0, 0].T * s[None, :]
    b1 = fp["fu_bn_b"].reshape(1, 256)
    wo = fp["fu2_w"][:, :, 0, 0].T
    bo = fp["fu2_b"].reshape(1, 128)

    return _fusion(bev3, bev4, bev5,
                   wcat[0:128], wcat[128:256], wcat[256:384], b1, wo, bo)
